# baseline (speedup 1.0000x reference)
import functools

import jax
import jax.numpy as jnp
from jax import lax
from jax.experimental import pallas as pl
from jax.experimental.pallas import tpu as pltpu
from jax.experimental.pallas import tpu_sc as plsc

_F = 26
_V = 100000
_D = 32
_B = 4096
_DX = 13
_DXP = 16
_NC, _NS = 2, 16
_NW = _NC * _NS
_BPW = _B // _NW
_RPB = 4
_CHUNK = _RPB * _F
_GROUPS = _BPW // 16
_IPW = _BPW * _F


def _rot(x, idx):
    return lax.gather(
        x, idx[:, None],
        dimension_numbers=lax.GatherDimensionNumbers(
            offset_dims=(), collapsed_slice_dims=(0,), start_index_map=(0,)),
        slice_sizes=(1,),
        mode=lax.GatherScatterMode.PROMISE_IN_BOUNDS)


def _fm_body(idx_hbm, offs_hbm, dx_hbm, w_hbm, b_hbm, tab_hbm, out_hbm,
             idx_v, offs_v, rows_v, dx_v, w_v, b_v, out_v, sem):
    wid = lax.axis_index("s") * _NC + lax.axis_index("c")
    base = wid * _BPW
    pltpu.sync_copy(idx_hbm.at[pl.ds(base * _F, _IPW)], idx_v)
    pltpu.sync_copy(offs_hbm, offs_v)
    pltpu.sync_copy(dx_hbm.at[pl.ds(base * _DXP, _BPW * _DXP)], dx_v)
    pltpu.sync_copy(w_hbm, w_v)
    pltpu.sync_copy(b_hbm, b_v)

    def add_offs(i, carry):
        sl = pl.ds(i * 16, 16)
        idx_v[sl] = idx_v[sl] + offs_v[sl]
        return carry

    lax.fori_loop(0, _IPW // 16, add_offs, 0)

    lanes = lax.iota(jnp.int32, 16)
    bias0 = b_v[pl.ds(0, 16)]
    bias1 = b_v[pl.ds(16, 16)]

    def group_body(g, carry):
        acc = jnp.zeros((16,), jnp.float32)
        for cc in range(16 // _RPB):
            c = g * (16 // _RPB) + cc
            pltpu.async_copy(
                tab_hbm.at[idx_v.at[pl.ds(c * _CHUNK, _CHUNK)]], rows_v, sem
            ).wait()
            for r in range(_RPB):
                rb = c * _RPB + r
                lane = cc * _RPB + r

                def fold(f, sq):
                    s0, s1, q0, q1 = sq
                    j = r * _F + f
                    v0 = rows_v[j, pl.ds(0, 16)]
                    v1 = rows_v[j, pl.ds(16, 16)]
                    return (s0 + v0, s1 + v1, q0 + v0 * v0, q1 + v1 * v1)

                z = jnp.zeros((16,), jnp.float32)
                s0, s1, q0, q1 = lax.fori_loop(0, _F, fold, (z, z, z, z))

                xv = dx_v[pl.ds(rb * _DXP, _DXP)]
                d0, d1 = bias0, bias1
                for k in range(_DX):
                    x = xv[k]
                    d0 = d0 + x * w_v[pl.ds(k * _D, 16)]
                    d1 = d1 + x * w_v[pl.ds(k * _D + 16, 16)]
                t0 = s0 + d0
                t1 = s1 + d1
                u = (t0 * t0 - q0 - d0 * d0) + (t1 * t1 - q1 - d1 * d1)
                red = u * 0.5
                for sh in (8, 4, 2, 1):
                    red = red + _rot(red, (lanes + sh) % 16)
                acc = jnp.where(lanes == lane, red, acc)
        out_v[pl.ds(g * 16, 16)] = acc
        return carry

    lax.fori_loop(0, _GROUPS, group_body, 0)
    pltpu.sync_copy(out_v, out_hbm.at[pl.ds(base, _BPW)])


@functools.partial(
    pl.kernel,
    out_type=jax.ShapeDtypeStruct((_B,), jnp.float32),
    mesh=plsc.VectorSubcoreMesh(core_axis_name="c", subcore_axis_name="s"),
    compiler_params=pltpu.CompilerParams(use_tc_tiling_on_sc=False),
    scratch_types=[
        pltpu.VMEM((_IPW,), jnp.int32),
        pltpu.VMEM((_IPW,), jnp.int32),
        pltpu.VMEM((_CHUNK, _D), jnp.float32),
        pltpu.VMEM((_BPW * _DXP,), jnp.float32),
        pltpu.VMEM((_DX * _D,), jnp.float32),
        pltpu.VMEM((_D,), jnp.float32),
        pltpu.VMEM((_BPW,), jnp.float32),
        pltpu.SemaphoreType.DMA,
    ],
)
def _fm_sc(*refs):
    _fm_body(*refs)


def kernel(dense_x, discrete_x, tables, W, b):
    flat_tab = tables.reshape(_F * _V, _D)
    idx_raw = discrete_x.astype(jnp.int32).reshape(-1)
    offs = jnp.tile(jnp.arange(_F, dtype=jnp.int32) * _V, _BPW)
    dx_pad = jnp.pad(dense_x, ((0, 0), (0, _DXP - _DX))).reshape(-1)
    return _fm_sc(idx_raw, offs, dx_pad, W.reshape(-1), b, flat_tab)


# speedup vs baseline: 1.0191x; 1.0191x over previous
import functools

import jax
import jax.numpy as jnp
from jax import lax
from jax.experimental import pallas as pl
from jax.experimental.pallas import tpu as pltpu
from jax.experimental.pallas import tpu_sc as plsc

_F = 26
_V = 100000
_D = 32
_B = 4096
_DX = 13
_DXP = 16
_NC, _NS = 2, 16
_NW = _NC * _NS
_BPW = _B // _NW
_RPB = 4
_CHUNK = _RPB * _F
_NCHUNK = _BPW // _RPB
_GROUPS = _BPW // 16
_IPW = _BPW * _F


def _rot(x, idx):
    return lax.gather(
        x, idx[:, None],
        dimension_numbers=lax.GatherDimensionNumbers(
            offset_dims=(), collapsed_slice_dims=(0,), start_index_map=(0,)),
        slice_sizes=(1,),
        mode=lax.GatherScatterMode.PROMISE_IN_BOUNDS)


def _fm_body(idx_hbm, offs_hbm, dx_hbm, w_hbm, b_hbm, tab_hbm, out_hbm,
             idx_v, offs_v, rows0, rows1, dx_v, w_v, b_v, out_v, sem0, sem1):
    wid = lax.axis_index("s") * _NC + lax.axis_index("c")
    base = wid * _BPW
    pltpu.sync_copy(idx_hbm.at[pl.ds(base * _F, _IPW)], idx_v)
    pltpu.sync_copy(offs_hbm, offs_v)
    pltpu.sync_copy(dx_hbm.at[pl.ds(base * _DXP, _BPW * _DXP)], dx_v)
    pltpu.sync_copy(w_hbm, w_v)
    pltpu.sync_copy(b_hbm, b_v)

    def add_offs(i, carry):
        for u in range(4):
            sl = pl.ds((i * 4 + u) * 16, 16)
            idx_v[sl] = idx_v[sl] + offs_v[sl]
        return carry

    lax.fori_loop(0, _IPW // 64, add_offs, 0)

    lanes = lax.iota(jnp.int32, 16)
    bias0 = b_v[pl.ds(0, 16)]
    bias1 = b_v[pl.ds(16, 16)]
    bufs = (rows0, rows1)
    sems = (sem0, sem1)

    def gather(c, p):
        pltpu.async_copy(
            tab_hbm.at[idx_v.at[pl.ds(c * _CHUNK, _CHUNK)]], bufs[p], sems[p])

    def gather_wait(c, p):
        pltpu.make_async_copy(
            tab_hbm.at[idx_v.at[pl.ds(c * _CHUNK, _CHUNK)]], bufs[p], sems[p]
        ).wait()

    gather(0, 0)

    def group_body(g, carry):
        acc = jnp.zeros((16,), jnp.float32)
        for cc in range(_NCHUNK // _GROUPS):
            c = g * (_NCHUNK // _GROUPS) + cc
            p = cc % 2

            @pl.when(c + 1 < _NCHUNK)
            def _():
                gather(c + 1, 1 - p)

            gather_wait(c, p)
            rows = bufs[p]
            for r in range(_RPB):
                rb = c * _RPB + r
                lane = cc * _RPB + r
                j = r * _F
                s0 = rows[j, pl.ds(0, 16)]
                s1 = rows[j, pl.ds(16, 16)]
                q0 = s0 * s0
                q1 = s1 * s1
                for f in range(1, _F):
                    v0 = rows[j + f, pl.ds(0, 16)]
                    v1 = rows[j + f, pl.ds(16, 16)]
                    s0 = s0 + v0
                    q0 = q0 + v0 * v0
                    s1 = s1 + v1
                    q1 = q1 + v1 * v1

                xv = dx_v[pl.ds(rb * _DXP, _DXP)]
                d0, d1 = bias0, bias1
                for k in range(_DX):
                    x = xv[k]
                    d0 = d0 + x * w_v[pl.ds(k * _D, 16)]
                    d1 = d1 + x * w_v[pl.ds(k * _D + 16, 16)]
                t0 = s0 + d0
                t1 = s1 + d1
                u = (t0 * t0 - q0 - d0 * d0) + (t1 * t1 - q1 - d1 * d1)
                red = u * 0.5
                for sh in (8, 4, 2, 1):
                    red = red + _rot(red, (lanes + sh) % 16)
                acc = jnp.where(lanes == lane, red, acc)
        out_v[pl.ds(g * 16, 16)] = acc
        return carry

    lax.fori_loop(0, _GROUPS, group_body, 0)
    pltpu.sync_copy(out_v, out_hbm.at[pl.ds(base, _BPW)])


@functools.partial(
    pl.kernel,
    out_type=jax.ShapeDtypeStruct((_B,), jnp.float32),
    mesh=plsc.VectorSubcoreMesh(core_axis_name="c", subcore_axis_name="s"),
    compiler_params=pltpu.CompilerParams(use_tc_tiling_on_sc=False),
    scratch_types=[
        pltpu.VMEM((_IPW,), jnp.int32),
        pltpu.VMEM((_IPW,), jnp.int32),
        pltpu.VMEM((_CHUNK, _D), jnp.float32),
        pltpu.VMEM((_CHUNK, _D), jnp.float32),
        pltpu.VMEM((_BPW * _DXP,), jnp.float32),
        pltpu.VMEM((_DX * _D,), jnp.float32),
        pltpu.VMEM((_D,), jnp.float32),
        pltpu.VMEM((_BPW,), jnp.float32),
        pltpu.SemaphoreType.DMA,
        pltpu.SemaphoreType.DMA,
    ],
)
def _fm_sc(*refs):
    _fm_body(*refs)


def kernel(dense_x, discrete_x, tables, W, b):
    flat_tab = tables.reshape(_F * _V, _D)
    idx_raw = discrete_x.astype(jnp.int32).reshape(-1)
    offs = jnp.tile(jnp.arange(_F, dtype=jnp.int32) * _V, _BPW)
    dx_pad = jnp.pad(dense_x, ((0, 0), (0, _DXP - _DX))).reshape(-1)
    return _fm_sc(idx_raw, offs, dx_pad, W.reshape(-1), b, flat_tab)
